# speedup vs baseline: 12.1084x; 12.1084x over previous
import jax
import jax.numpy as jnp
from jax.experimental import pallas as pl

_BB = 256


def _bcast_kernel(pe_ref, out_ref):
    out_ref[...] = jnp.broadcast_to(pe_ref[...], out_ref.shape)


def kernel(x, pe):
    batch, seq_len = x.shape
    max_len, d_model = pe.shape
    flat = seq_len * d_model
    pe_flat = pe.reshape(1, flat)

    out = pl.pallas_call(
        _bcast_kernel,
        grid=(batch // _BB,),
        in_specs=[pl.BlockSpec((1, flat), lambda i: (0, 0))],
        out_specs=pl.BlockSpec((_BB, flat), lambda i: (i, 0)),
        out_shape=jax.ShapeDtypeStruct((batch, flat), jnp.float32),
    )(pe_flat)
    return out.reshape(batch, seq_len, d_model)
